# baseline (speedup 1.0000x reference)
import functools
import math

import jax
import jax.numpy as jnp
from jax import lax
from jax.experimental import pallas as pl
from jax.experimental.pallas import tpu as pltpu
from jax.experimental.pallas import tpu_sc as plsc

SIZE = 32000
N_ROWS = 4096
IGNORE_IDX = -100
SMOOTHING = 0.1
CONFIDENCE = 1.0 - SMOOTHING
IGN_COL = SIZE + IGNORE_IDX
EPS = SMOOTHING / (SIZE - 2)
_LOG_EPS = math.log(EPS)
ENT_A = (SIZE - 2) * EPS * _LOG_EPS + CONFIDENCE * math.log(CONFIDENCE)
ENT_B = (SIZE - 1) * EPS * _LOG_EPS

LANES = 16
NUM_WORKERS = 32
RPW = N_ROWS // NUM_WORKERS


TILE_R = 8
TILE_C = 128


def _sc_body(src2, trg_hbm, p_out, trg_v, dst0_v, dst1_v, psum_v, sem0, sem1):
    wid = lax.axis_index("s") * 2 + lax.axis_index("c")
    base = wid * RPW
    pltpu.sync_copy(trg_hbm.at[pl.ds(base, RPW)], trg_v)
    iota = lax.iota(jnp.int32, LANES)
    acc = jnp.zeros((LANES,), jnp.float32)
    dsts = (dst0_v, dst1_v)
    sems = (sem0, sem1)
    nb = RPW // LANES
    t_b, lj_b, cps_b = {}, {}, {}

    def fire(k):
        t_raw = trg_v[pl.ds(k * LANES, LANES)]
        t = jnp.maximum(t_raw, 0)
        c0s = (t >> 7) * TILE_C
        lj_b[k] = t & (TILE_C - 1)
        t_b[k] = t_raw
        dst = dsts[k % 2]
        cps = []
        for j in range(LANES):
            c0 = pl.multiple_of(c0s[j], TILE_C)
            row_t = base + k * LANES + (j & ~(TILE_R - 1))
            cps.append(pltpu.async_copy(
                src2.at[pl.ds(row_t, TILE_R), pl.ds(c0, TILE_C)],
                dst.at[pl.ds(j * TILE_R, TILE_R), :],
                sems[k % 2]))
        cps_b[k] = cps

    fire(0)
    for k in range(nb):
        if k + 1 < nb:
            fire(k + 1)
        for cp in cps_b[k]:
            cp.wait()
        dst = dsts[k % 2]
        lj = lj_b[k]
        sel = jnp.zeros((LANES,), jnp.float32)
        for j in range(LANES):
            s = lj[j] - j
            v = dst[j * TILE_R + (j & (TILE_R - 1)), pl.ds(s, LANES)]
            sel = jnp.where(iota == j, v, sel)
        keep = (t_b[k] != IGN_COL) & (t_b[k] != IGNORE_IDX)
        acc = acc + jnp.where(keep, sel, 0.0)
    psum_v[...] = acc
    pltpu.sync_copy(psum_v, p_out.at[pl.ds(wid * LANES, LANES)])


def _sc_gather_sum(src, trg32):
    mesh = plsc.VectorSubcoreMesh(core_axis_name="c", subcore_axis_name="s")
    f = functools.partial(
        pl.kernel,
        mesh=mesh,
        out_type=jax.ShapeDtypeStruct((NUM_WORKERS * LANES,), jnp.float32),
        scratch_types=[
            pltpu.VMEM((RPW,), jnp.int32),
            pltpu.VMEM((LANES * TILE_R, TILE_C), jnp.float32),
            pltpu.VMEM((LANES * TILE_R, TILE_C), jnp.float32),
            pltpu.VMEM((LANES,), jnp.float32),
            pltpu.SemaphoreType.DMA,
            pltpu.SemaphoreType.DMA,
        ],
    )(_sc_body)
    return f(src, trg32)


BR = 128
GR = N_ROWS // BR


def _rowsum_body(src_ref, rs_ref, cv_ref):
    rs_ref[...] = jnp.sum(src_ref[...], axis=1, keepdims=True)
    cv_ref[...] = src_ref[:, IGN_COL:IGN_COL + 1]


def _rowsum(src):
    return pl.pallas_call(
        _rowsum_body,
        grid=(GR,),
        in_specs=[pl.BlockSpec((BR, SIZE), lambda r: (r, 0))],
        out_specs=[
            pl.BlockSpec((BR, 1), lambda r: (r, 0)),
            pl.BlockSpec((BR, 1), lambda r: (r, 0)),
        ],
        out_shape=[
            jax.ShapeDtypeStruct((N_ROWS, 1), jnp.float32),
            jax.ShapeDtypeStruct((N_ROWS, 1), jnp.float32),
        ],
    )(src)


CR = 32
CC = N_ROWS // CR


def _combine_body(rs_ref, cv_ref, trg_ref, gsum_ref, out_ref):
    cv = cv_ref[...]
    t = trg_ref[...]
    w_a = ENT_A + EPS * cv
    w_b = ENT_B + EPS * cv
    w = jnp.where(t == IGN_COL, w_b, w_a)
    w = jnp.where(t == IGNORE_IDX, 0.0, w)
    total = (jnp.sum(w)
             + (EPS - CONFIDENCE) * jnp.sum(gsum_ref[...])
             - EPS * jnp.sum(rs_ref[...]))
    out_ref[...] = (total / N_ROWS).reshape(1, 1)


def _combine(rs, cv, trg32, gsum):
    return pl.pallas_call(
        _combine_body,
        out_shape=jax.ShapeDtypeStruct((1, 1), jnp.float32),
    )(rs.reshape(CR, CC), cv.reshape(CR, CC), trg32.reshape(CR, CC),
      gsum.reshape(4, 128))


def kernel(src, trg):
    trg32 = trg.astype(jnp.int32)
    gsum = _sc_gather_sum(src, trg32)
    rs, cv = _rowsum(src)
    return _combine(rs, cv, trg32, gsum)[0, 0]


# speedup vs baseline: 1.0032x; 1.0032x over previous
import functools
import math

import jax
import jax.numpy as jnp
from jax import lax
from jax.experimental import pallas as pl
from jax.experimental.pallas import tpu as pltpu
from jax.experimental.pallas import tpu_sc as plsc

SIZE = 32000
N_ROWS = 4096
IGNORE_IDX = -100
SMOOTHING = 0.1
CONFIDENCE = 1.0 - SMOOTHING
IGN_COL = SIZE + IGNORE_IDX
EPS = SMOOTHING / (SIZE - 2)
_LOG_EPS = math.log(EPS)
ENT_A = (SIZE - 2) * EPS * _LOG_EPS + CONFIDENCE * math.log(CONFIDENCE)
ENT_B = (SIZE - 1) * EPS * _LOG_EPS

LANES = 16
NUM_WORKERS = 32
RPW = N_ROWS // NUM_WORKERS


TILE_R = 8
TILE_C = 128


def _sc_body(src2, trg_hbm, p_out, trg_v, dst0_v, dst1_v, psum_v, sem0, sem1):
    wid = lax.axis_index("s") * 2 + lax.axis_index("c")
    base = wid * RPW
    pltpu.sync_copy(trg_hbm.at[pl.ds(base, RPW)], trg_v)
    iota = lax.iota(jnp.int32, LANES)
    acc = jnp.zeros((LANES,), jnp.float32)
    dsts = (dst0_v, dst1_v)
    sems = (sem0, sem1)
    nb = RPW // LANES
    t_b, lj_b, cps_b = {}, {}, {}

    def fire(k):
        t_raw = trg_v[pl.ds(k * LANES, LANES)]
        t = jnp.maximum(t_raw, 0)
        c0s = (t >> 7) * TILE_C
        lj_b[k] = t & (TILE_C - 1)
        t_b[k] = t_raw
        dst = dsts[k % 2]
        cps = []
        for j in range(LANES):
            c0 = pl.multiple_of(c0s[j], TILE_C)
            row_t = base + k * LANES + (j & ~(TILE_R - 1))
            cps.append(pltpu.async_copy(
                src2.at[pl.ds(row_t, TILE_R), pl.ds(c0, TILE_C)],
                dst.at[pl.ds(j * TILE_R, TILE_R), :],
                sems[k % 2]))
        cps_b[k] = cps

    fire(0)
    for k in range(nb):
        if k + 1 < nb:
            fire(k + 1)
        for cp in cps_b[k]:
            cp.wait()
        dst = dsts[k % 2]
        lj = lj_b[k]
        sel = jnp.zeros((LANES,), jnp.float32)
        for j in range(LANES):
            s = lj[j] - j
            v = dst[j * TILE_R + (j & (TILE_R - 1)), pl.ds(s, LANES)]
            sel = jnp.where(iota == j, v, sel)
        keep = (t_b[k] != IGN_COL) & (t_b[k] != IGNORE_IDX)
        acc = acc + jnp.where(keep, sel, 0.0)
    psum_v[...] = acc
    pltpu.sync_copy(psum_v, p_out.at[pl.ds(wid * LANES, LANES)])


def _sc_gather_sum(src, trg32):
    mesh = plsc.VectorSubcoreMesh(core_axis_name="c", subcore_axis_name="s")
    f = functools.partial(
        pl.kernel,
        mesh=mesh,
        out_type=jax.ShapeDtypeStruct((NUM_WORKERS * LANES,), jnp.float32),
        scratch_types=[
            pltpu.VMEM((RPW,), jnp.int32),
            pltpu.VMEM((LANES * TILE_R, TILE_C), jnp.float32),
            pltpu.VMEM((LANES * TILE_R, TILE_C), jnp.float32),
            pltpu.VMEM((LANES,), jnp.float32),
            pltpu.SemaphoreType.DMA,
            pltpu.SemaphoreType.DMA,
        ],
    )(_sc_body)
    return f(src, trg32)


BR = 64
GR = N_ROWS // BR


def _rowsum_body(src_ref, rs_ref, cv_ref):
    rs_ref[...] = jnp.sum(src_ref[...], axis=1, keepdims=True)
    cv_ref[...] = src_ref[:, IGN_COL:IGN_COL + 1]


def _rowsum(src):
    return pl.pallas_call(
        _rowsum_body,
        grid=(GR,),
        in_specs=[pl.BlockSpec((BR, SIZE), lambda r: (r, 0))],
        out_specs=[
            pl.BlockSpec((BR, 1), lambda r: (r, 0)),
            pl.BlockSpec((BR, 1), lambda r: (r, 0)),
        ],
        out_shape=[
            jax.ShapeDtypeStruct((N_ROWS, 1), jnp.float32),
            jax.ShapeDtypeStruct((N_ROWS, 1), jnp.float32),
        ],
    )(src)


CR = 32
CC = N_ROWS // CR


def _combine_body(rs_ref, cv_ref, trg_ref, gsum_ref, out_ref):
    cv = cv_ref[...]
    t = trg_ref[...]
    w_a = ENT_A + EPS * cv
    w_b = ENT_B + EPS * cv
    w = jnp.where(t == IGN_COL, w_b, w_a)
    w = jnp.where(t == IGNORE_IDX, 0.0, w)
    total = (jnp.sum(w)
             + (EPS - CONFIDENCE) * jnp.sum(gsum_ref[...])
             - EPS * jnp.sum(rs_ref[...]))
    out_ref[...] = (total / N_ROWS).reshape(1, 1)


def _combine(rs, cv, trg32, gsum):
    return pl.pallas_call(
        _combine_body,
        out_shape=jax.ShapeDtypeStruct((1, 1), jnp.float32),
    )(rs.reshape(CR, CC), cv.reshape(CR, CC), trg32.reshape(CR, CC),
      gsum.reshape(4, 128))


def kernel(src, trg):
    trg32 = trg.astype(jnp.int32)
    gsum = _sc_gather_sum(src, trg32)
    rs, cv = _rowsum(src)
    return _combine(rs, cv, trg32, gsum)[0, 0]
